# Initial kernel scaffold; baseline (speedup 1.0000x reference)
import jax
import jax.numpy as jnp
from jax.experimental import pallas as pl


def kernel(x):
    raise NotImplementedError("write your pallas kernel here")


# speedup vs baseline: 1.1379x; 1.1379x over previous
import functools

import jax
import jax.numpy as jnp
from jax import lax
from jax.experimental import pallas as pl
from jax.experimental.pallas import tpu as pltpu
from jax.experimental.pallas import tpu_sc as plsc

K = 32
L = 16
ROWS = 128
N = 32768
NCHUNKS = N // L
CAP = 4096
NW = 32
ROWS_PER_W = ROWS // NW


def _sort_asc(v):
    return plsc.sort_key_val(v, v)[0]


def _sort_desc(v):
    return plsc.sort_key_val(v, v, descending=True)[0]


def _merge(H1, H2, c):
    hi = jnp.maximum(_sort_asc(H2), _sort_desc(c))
    s_h1 = _sort_asc(H1)
    s_hi = _sort_desc(hi)
    H1n = jnp.maximum(s_h1, s_hi)
    H2n = jnp.minimum(s_h1, s_hi)
    thetaSn = jnp.broadcast_to(jnp.min(H2n), (L,))
    return H1n, H2n, thetaSn


def _body(x_hbm, out_hbm, rowbuf, outrow, bufV, bufI, outIbuf, off_ref):
    wid = lax.axis_index("c") * 16 + lax.axis_index("s")
    zeros16 = jnp.zeros((L,), jnp.float32)
    iota16 = lax.broadcasted_iota(jnp.int32, (L,), 0)

    def _z(kk, _):
        outrow[pl.ds(kk * L, L)] = zeros16
        return 0
    lax.fori_loop(0, NCHUNKS, _z, 0)

    def row_body(r, _):
        rowbase = (wid * ROWS_PER_W + r) * N
        pltpu.sync_copy(x_hbm.at[pl.ds(rowbase, N)], rowbuf)

        c0 = rowbuf[pl.ds(0, L)]
        c1 = rowbuf[pl.ds(L, L)]
        s0 = _sort_asc(c0)
        s1 = _sort_desc(c1)
        H1 = jnp.maximum(s0, s1)
        H2 = jnp.minimum(s0, s1)
        thetaS = jnp.broadcast_to(jnp.min(H2), (L,))
        bufV[pl.ds(0, L)] = c0
        bufI[pl.ds(0, L)] = iota16
        bufV[pl.ds(L, L)] = c1
        bufI[pl.ds(L, L)] = iota16 + L
        off_ref[0] = 2 * L

        def chunk_body(i, carry):
            H1, H2, thetaS = carry
            c = rowbuf[pl.ds(i * L, L)]
            hit = jnp.any(c > thetaS)

            @pl.when(hit)
            def _append():
                off = off_ref[0]

                @pl.when(off >= CAP - L)
                def _compact():
                    noff = off_ref[0]

                    def comp_body(j, w):
                        cv = bufV[pl.ds(j * L, L)]
                        keep = jnp.any(cv >= thetaS)

                        @pl.when(keep)
                        def _wr():
                            bufV[pl.ds(w, L)] = cv
                            bufI[pl.ds(w, L)] = bufI[pl.ds(j * L, L)]
                        return w + jnp.where(keep, L, 0)

                    w = lax.fori_loop(0, noff // L, comp_body, 0)
                    off_ref[0] = w

                off2 = off_ref[0]
                bufV[pl.ds(off2, L)] = c
                bufI[pl.ds(off2, L)] = iota16 + i * L
                off_ref[0] = off2 + L

            H1, H2, thetaS = lax.cond(
                hit, _merge, lambda a, b, c_: (a, b, thetaS), H1, H2, c)
            return H1, H2, thetaS

        H1, H2, thetaS = lax.fori_loop(
            2, NCHUNKS, chunk_body, (H1, H2, thetaS))
        tS = thetaS

        nbc = off_ref[0] // L

        def gt_body(j, acc):
            v = bufV[pl.ds(j * L, L)]
            return acc + jnp.sum((v > tS).astype(jnp.int32))

        gt_total = lax.fori_loop(0, nbc, gt_body, jnp.int32(0))
        need = K - gt_total

        def sel_body(j, carry):
            gpos, eqb = carry
            v = bufV[pl.ds(j * L, L)]
            ii = bufI[pl.ds(j * L, L)]
            m_gt = v > tS
            m_eqr = v == tS
            r_gt = plsc.cumsum(m_gt.astype(jnp.int32))
            r_eq = plsc.cumsum(m_eqr.astype(jnp.int32))
            keep_eq = m_eqr & (eqb + r_eq <= need)
            keep = m_gt | keep_eq
            plsc.store_scatter(outrow, [ii], jnp.maximum(v, 0.0), mask=keep)
            pos = jnp.where(m_gt, gpos + r_gt - 1, K - (eqb + r_eq))
            pos = jnp.minimum(jnp.maximum(pos, 0), K - 1)
            plsc.store_scatter(outIbuf, [pos], ii, mask=keep)
            return (gpos + jnp.sum(m_gt.astype(jnp.int32)),
                    eqb + jnp.sum(m_eqr.astype(jnp.int32)))

        lax.fori_loop(0, nbc, sel_body, (jnp.int32(0), jnp.int32(0)))

        pltpu.sync_copy(outrow, out_hbm.at[pl.ds(rowbase, N)])
        i0 = outIbuf[pl.ds(0, L)]
        i1 = outIbuf[pl.ds(L, L)]
        plsc.store_scatter(outrow, [i0], zeros16)
        plsc.store_scatter(outrow, [i1], zeros16)
        return 0

    lax.fori_loop(0, ROWS_PER_W, row_body, 0)


def kernel(x):
    mesh = plsc.VectorSubcoreMesh(core_axis_name="c", subcore_axis_name="s")
    f = functools.partial(
        pl.kernel,
        mesh=mesh,
        compiler_params=pltpu.CompilerParams(needs_layout_passes=False),
        out_type=jax.ShapeDtypeStruct((ROWS * N,), jnp.float32),
        scratch_types=[
            pltpu.VMEM((N,), jnp.float32),
            pltpu.VMEM((N,), jnp.float32),
            pltpu.VMEM((CAP,), jnp.float32),
            pltpu.VMEM((CAP,), jnp.int32),
            pltpu.VMEM((K,), jnp.int32),
            pltpu.SMEM((1,), jnp.int32),
        ],
    )(_body)
    out = f(x.reshape(-1))
    return out.reshape(ROWS, N)


# speedup vs baseline: 1.8223x; 1.6015x over previous
import functools

import jax
import jax.numpy as jnp
from jax import lax
from jax.experimental import pallas as pl
from jax.experimental.pallas import tpu as pltpu
from jax.experimental.pallas import tpu_sc as plsc

K = 32
L = 16
ROWS = 128
N = 32768
NCHUNKS = N // L
CAP = 4096
NW = 32
ROWS_PER_W = ROWS // NW


def _sort_asc(v):
    return plsc.sort_key_val(v, v)[0]


def _sort_desc(v):
    return plsc.sort_key_val(v, v, descending=True)[0]


def _merge(H1, H2, c):
    hi = jnp.maximum(_sort_asc(H2), _sort_desc(c))
    s_h1 = _sort_asc(H1)
    s_hi = _sort_desc(hi)
    H1n = jnp.maximum(s_h1, s_hi)
    H2n = jnp.minimum(s_h1, s_hi)
    thetaSn = jnp.broadcast_to(jnp.min(H2n), (L,))
    return H1n, H2n, thetaSn


def _body(x_hbm, out_hbm, rowbuf, outrow, bufV, bufI, outIbuf, off_ref):
    wid = lax.axis_index("c") * 16 + lax.axis_index("s")
    zeros16 = jnp.zeros((L,), jnp.float32)
    iota16 = lax.broadcasted_iota(jnp.int32, (L,), 0)

    def _z(kk, _):
        outrow[pl.ds(kk * L, L)] = zeros16
        return 0
    lax.fori_loop(0, NCHUNKS, _z, 0)

    def row_body(r, _):
        rowbase = (wid * ROWS_PER_W + r) * N
        pltpu.sync_copy(x_hbm.at[pl.ds(rowbase, N)], rowbuf)

        c0 = rowbuf[pl.ds(0, L)]
        c1 = rowbuf[pl.ds(L, L)]
        s0 = _sort_asc(c0)
        s1 = _sort_desc(c1)
        H1 = jnp.maximum(s0, s1)
        H2 = jnp.minimum(s0, s1)
        thetaS = jnp.broadcast_to(jnp.min(H2), (L,))
        bufV[pl.ds(0, L)] = c0
        bufI[pl.ds(0, L)] = iota16
        bufV[pl.ds(L, L)] = c1
        bufI[pl.ds(L, L)] = iota16 + L
        off_ref[0] = 2 * L

        def handle_chunk(c, base_i, carry):
            H1, H2, thetaS = carry
            hit = jnp.any(c > thetaS)

            def _hitfn(H1, H2, thetaS):
                off = off_ref[0]

                @pl.when(off >= CAP - L)
                def _compact():
                    noff = off_ref[0]

                    def comp_body(j, w):
                        cv = bufV[pl.ds(j * L, L)]
                        keep = jnp.any(cv >= thetaS)

                        @pl.when(keep)
                        def _wr():
                            bufV[pl.ds(w, L)] = cv
                            bufI[pl.ds(w, L)] = bufI[pl.ds(j * L, L)]
                        return w + jnp.where(keep, L, 0)

                    w = lax.fori_loop(0, noff // L, comp_body, 0)
                    off_ref[0] = w

                off2 = off_ref[0]
                bufV[pl.ds(off2, L)] = c
                bufI[pl.ds(off2, L)] = iota16 + base_i
                off_ref[0] = off2 + L
                return _merge(H1, H2, c)

            return lax.cond(
                hit, _hitfn, lambda a, b, c_: (a, b, c_), H1, H2, thetaS)

        carry = (H1, H2, thetaS)
        for i in (2, 3):
            carry = handle_chunk(rowbuf[pl.ds(i * L, L)], i * L, carry)

        def group_body(g, carry):
            base = g * 4 * L
            c0 = rowbuf[pl.ds(base, L)]
            c1 = rowbuf[pl.ds(base + L, L)]
            c2 = rowbuf[pl.ds(base + 2 * L, L)]
            c3 = rowbuf[pl.ds(base + 3 * L, L)]
            mx = jnp.maximum(jnp.maximum(c0, c1), jnp.maximum(c2, c3))
            ghit = jnp.any(mx > carry[2])

            def _gfn(H1, H2, thetaS):
                cr = (H1, H2, thetaS)
                cr = handle_chunk(c0, base, cr)
                cr = handle_chunk(c1, base + L, cr)
                cr = handle_chunk(c2, base + 2 * L, cr)
                cr = handle_chunk(c3, base + 3 * L, cr)
                return cr

            return lax.cond(
                ghit, _gfn, lambda a, b, c_: (a, b, c_), *carry)

        H1, H2, thetaS = lax.fori_loop(1, NCHUNKS // 4, group_body, carry)
        tS = thetaS

        nbc = off_ref[0] // L

        def gt_body(j, acc):
            v = bufV[pl.ds(j * L, L)]
            return acc + jnp.sum((v > tS).astype(jnp.int32))

        gt_total = lax.fori_loop(0, nbc, gt_body, jnp.int32(0))
        need = K - gt_total

        def sel_body(j, carry):
            gpos, eqb = carry
            v = bufV[pl.ds(j * L, L)]
            ii = bufI[pl.ds(j * L, L)]
            m_gt = v > tS
            m_eqr = v == tS
            r_gt = plsc.cumsum(m_gt.astype(jnp.int32))
            r_eq = plsc.cumsum(m_eqr.astype(jnp.int32))
            keep_eq = m_eqr & (eqb + r_eq <= need)
            keep = m_gt | keep_eq
            plsc.store_scatter(outrow, [ii], jnp.maximum(v, 0.0), mask=keep)
            pos = jnp.where(m_gt, gpos + r_gt - 1, K - (eqb + r_eq))
            pos = jnp.minimum(jnp.maximum(pos, 0), K - 1)
            plsc.store_scatter(outIbuf, [pos], ii, mask=keep)
            return (gpos + jnp.sum(m_gt.astype(jnp.int32)),
                    eqb + jnp.sum(m_eqr.astype(jnp.int32)))

        lax.fori_loop(0, nbc, sel_body, (jnp.int32(0), jnp.int32(0)))

        pltpu.sync_copy(outrow, out_hbm.at[pl.ds(rowbase, N)])
        i0 = outIbuf[pl.ds(0, L)]
        i1 = outIbuf[pl.ds(L, L)]
        plsc.store_scatter(outrow, [i0], zeros16)
        plsc.store_scatter(outrow, [i1], zeros16)
        return 0

    lax.fori_loop(0, ROWS_PER_W, row_body, 0)


def kernel(x):
    mesh = plsc.VectorSubcoreMesh(core_axis_name="c", subcore_axis_name="s")
    f = functools.partial(
        pl.kernel,
        mesh=mesh,
        compiler_params=pltpu.CompilerParams(needs_layout_passes=False),
        out_type=jax.ShapeDtypeStruct((ROWS * N,), jnp.float32),
        scratch_types=[
            pltpu.VMEM((N,), jnp.float32),
            pltpu.VMEM((N,), jnp.float32),
            pltpu.VMEM((CAP,), jnp.float32),
            pltpu.VMEM((CAP,), jnp.int32),
            pltpu.VMEM((K,), jnp.int32),
            pltpu.SMEM((1,), jnp.int32),
        ],
    )(_body)
    out = f(x.reshape(-1))
    return out.reshape(ROWS, N)
